# baseline (speedup 1.0000x reference)
import jax
import jax.numpy as jnp
from jax import lax
from jax.experimental import pallas as pl
from jax.experimental.pallas import tpu as pltpu

B = 128
C = 100000
D = 128

BC = 8192
NB = (C + BC - 1) // BC
NG = 4
RG = B // NG


def _fused_body(ytc_ref, ytr_ref,
                x0_ref, x1_ref, x2_ref, x3_ref,
                w0_ref, w1_ref, w2_ref, w3_ref,
                o_ref,
                m_s, s_s, a_s, t_s, pr_s, wij_s):
    i = pl.program_id(0)
    x_refs = (x0_ref, x1_ref, x2_ref, x3_ref)
    w_refs = (w0_ref, w1_ref, w2_ref, w3_ref)

    @pl.when(i == 0)
    def _init():
        m_s[...] = jnp.full((B, 1), -jnp.inf, jnp.float32)
        s_s[...] = jnp.zeros((B, 1), jnp.float32)
        a_s[...] = jnp.zeros((B, 1), jnp.int32)
        t_s[...] = jnp.zeros((B, 1), jnp.float32)
        wij_s[...] = jnp.zeros((D, 2 * B), jnp.float32)
        pr_s[:, B:] = ytr_ref[...]

    def _stats_update(g, xm, gcol):
        lo = g * RG
        sl = slice(lo, lo + RG)
        m_old = m_s[sl]
        bm = jnp.max(xm, axis=1, keepdims=True)
        m_new = jnp.maximum(m_old, bm)
        s_s[sl] = s_s[sl] * jnp.exp(m_old - m_new) + jnp.sum(
            jnp.exp(xm - m_new), axis=1, keepdims=True)
        m_s[sl] = m_new

        cand = jnp.min(jnp.where(xm == bm, gcol, jnp.int32(2**30)),
                       axis=1, keepdims=True)
        a_s[sl] = jnp.where(bm > m_old, cand, a_s[sl])

        t_s[sl] = t_s[sl] + jnp.sum(
            jnp.where(gcol == ytc_ref[sl], xm, 0.0), axis=1, keepdims=True)

    @pl.when(i < NB - 1)
    def _phase_a_fast():
        gcol = i * BC + lax.broadcasted_iota(jnp.int32, (1, BC), 1)
        for g in range(NG):
            _stats_update(g, x_refs[g][...], gcol)

    @pl.when(i == NB - 1)
    def _phase_a_tail():
        gcol = i * BC + lax.broadcasted_iota(jnp.int32, (1, BC), 1)
        valid = gcol < C
        for g in range(NG):
            _stats_update(g, jnp.where(valid, x_refs[g][...], -jnp.inf),
                          gcol)

    @pl.when(i == NB)
    def _pred_to_row():
        eye = (lax.broadcasted_iota(jnp.int32, (B, B), 0) ==
               lax.broadcasted_iota(jnp.int32, (B, B), 1)).astype(jnp.float32)
        pr_s[:, :B] = lax.dot_general(
            a_s[...].astype(jnp.float32), eye, (((0,), (0,)), ((), ())),
            preferred_element_type=jnp.float32)

    def _gather_update(g, wb, oh):
        lo = g * RG
        wij_s[lo:lo + RG, :] = wij_s[lo:lo + RG, :] + lax.dot_general(
            wb, oh, (((1,), (0,)), ((), ())),
            preferred_element_type=jnp.float32)

    @pl.when(jnp.logical_and(i >= NB, i < 2 * NB - 1))
    def _phase_b_fast():
        j = i - NB
        gcolf = (j * BC +
                 lax.broadcasted_iota(jnp.int32, (BC, 1), 0)
                 ).astype(jnp.float32)
        oh = (gcolf == pr_s[...]).astype(jnp.bfloat16)
        for g in range(NG):
            _gather_update(g, w_refs[g][...].astype(jnp.bfloat16), oh)

    @pl.when(i == 2 * NB - 1)
    def _phase_b_tail():
        j = NB - 1
        gcol_c = j * BC + lax.broadcasted_iota(jnp.int32, (BC, 1), 0)
        oh = (gcol_c.astype(jnp.float32) == pr_s[...]).astype(jnp.bfloat16)
        valid = (j * BC + lax.broadcasted_iota(jnp.int32, (1, BC), 1)) < C
        for g in range(NG):
            wb = jnp.where(valid, w_refs[g][...], 0.0).astype(jnp.bfloat16)
            _gather_update(g, wb, oh)

        eye = (lax.broadcasted_iota(jnp.int32, (B, B), 0) ==
               lax.broadcasted_iota(jnp.int32, (B, B), 1)).astype(jnp.float32)
        wi = wij_s[:, :B]
        wj = wij_s[:, B:]
        mrow = jnp.sum(wi * wj, axis=0, keepdims=True)
        mcol = lax.dot_general(eye, mrow, (((1,), (1,)), ((), ())),
                               preferred_element_type=jnp.float32)
        m = m_s[...]
        t = t_s[...]
        zz = s_s[...] - jnp.exp(t - m) + jnp.exp(t - mcol - m)
        lossv = m + jnp.log(zz) - t + mcol
        o_ref[...] = jnp.sum(lossv, axis=0, keepdims=True) * (1.0 / B)


_fused = pl.pallas_call(
    _fused_body,
    grid=(2 * NB,),
    in_specs=(
        [pl.BlockSpec((B, 1), lambda i: (0, 0)),
         pl.BlockSpec((1, B), lambda i: (0, 0))] +
        [pl.BlockSpec((RG, BC), lambda i, g=g: (g, jnp.minimum(i, NB - 1)))
         for g in range(NG)] +
        [pl.BlockSpec((RG, BC), lambda i, g=g: (g, jnp.maximum(i - NB, 0)))
         for g in range(NG)]
    ),
    out_specs=pl.BlockSpec((1, 1), lambda i: (0, 0)),
    out_shape=jax.ShapeDtypeStruct((1, 1), jnp.float32),
    scratch_shapes=[
        pltpu.VMEM((B, 1), jnp.float32),
        pltpu.VMEM((B, 1), jnp.float32),
        pltpu.VMEM((B, 1), jnp.int32),
        pltpu.VMEM((B, 1), jnp.float32),
        pltpu.VMEM((1, 2 * B), jnp.float32),
        pltpu.VMEM((D, 2 * B), jnp.float32),
    ],
    compiler_params=pltpu.CompilerParams(
        dimension_semantics=("arbitrary",)),
)


@jax.jit
def kernel(y_pred, y_true, W):
    y_true = y_true.astype(jnp.int32)
    ytc = y_true.reshape(B, 1)
    ytr = y_true.astype(jnp.float32).reshape(1, B)
    loss = _fused(ytc, ytr,
                  y_pred, y_pred, y_pred, y_pred,
                  W, W, W, W)
    return loss.reshape(())


# speedup vs baseline: 1.0900x; 1.0900x over previous
import jax
import jax.numpy as jnp
from jax import lax
from jax.experimental import pallas as pl
from jax.experimental.pallas import tpu as pltpu

B = 128
C = 100000
D = 128

BC = 16384
NB = (C + BC - 1) // BC
NG = 1
RG = B // NG


def _fused_body(ytc_ref, ytr_ref, *args):
    x_refs = args[0:NG]
    w_refs = args[NG:2 * NG]
    o_ref = args[2 * NG]
    m_s, s_s, a_s, t_s, pr_s, wij_s = args[2 * NG + 1:]
    i = pl.program_id(0)

    @pl.when(i == 0)
    def _init():
        m_s[...] = jnp.full((B, 1), -jnp.inf, jnp.float32)
        s_s[...] = jnp.zeros((B, 1), jnp.float32)
        a_s[...] = jnp.zeros((B, 1), jnp.int32)
        t_s[...] = jnp.zeros((B, 1), jnp.float32)
        wij_s[...] = jnp.zeros((D, 2 * B), jnp.float32)
        pr_s[:, B:] = ytr_ref[...]

    def _stats_update(g, xm, gcol):
        lo = g * RG
        sl = slice(lo, lo + RG)
        m_old = m_s[sl]
        bm = jnp.max(xm, axis=1, keepdims=True)
        m_new = jnp.maximum(m_old, bm)
        s_s[sl] = s_s[sl] * jnp.exp(m_old - m_new) + jnp.sum(
            jnp.exp(xm - m_new), axis=1, keepdims=True)
        m_s[sl] = m_new

        cand = jnp.min(jnp.where(xm == bm, gcol, jnp.int32(2**30)),
                       axis=1, keepdims=True)
        a_s[sl] = jnp.where(bm > m_old, cand, a_s[sl])

        t_s[sl] = t_s[sl] + jnp.sum(
            jnp.where(gcol == ytc_ref[sl], xm, 0.0), axis=1, keepdims=True)

    @pl.when(i < NB - 1)
    def _phase_a_fast():
        gcol = i * BC + lax.broadcasted_iota(jnp.int32, (1, BC), 1)
        for g in range(NG):
            _stats_update(g, x_refs[g][...], gcol)

    @pl.when(i == NB - 1)
    def _phase_a_tail():
        gcol = i * BC + lax.broadcasted_iota(jnp.int32, (1, BC), 1)
        valid = gcol < C
        for g in range(NG):
            _stats_update(g, jnp.where(valid, x_refs[g][...], -jnp.inf),
                          gcol)

    @pl.when(i == NB)
    def _pred_to_row():
        eye = (lax.broadcasted_iota(jnp.int32, (B, B), 0) ==
               lax.broadcasted_iota(jnp.int32, (B, B), 1)).astype(jnp.float32)
        pr_s[:, :B] = lax.dot_general(
            a_s[...].astype(jnp.float32), eye, (((0,), (0,)), ((), ())),
            preferred_element_type=jnp.float32)

    def _gather_update(g, wb, oh):
        lo = g * RG
        wij_s[lo:lo + RG, :] = wij_s[lo:lo + RG, :] + lax.dot_general(
            wb, oh, (((1,), (0,)), ((), ())),
            preferred_element_type=jnp.float32)

    @pl.when(jnp.logical_and(i >= NB, i < 2 * NB - 1))
    def _phase_b_fast():
        j = i - NB
        gcolf = (j * BC +
                 lax.broadcasted_iota(jnp.int32, (BC, 1), 0)
                 ).astype(jnp.float32)
        oh = (gcolf == pr_s[...]).astype(jnp.bfloat16)
        for g in range(NG):
            _gather_update(g, w_refs[g][...].astype(jnp.bfloat16), oh)

    @pl.when(i == 2 * NB - 1)
    def _phase_b_tail():
        j = NB - 1
        gcol_c = j * BC + lax.broadcasted_iota(jnp.int32, (BC, 1), 0)
        oh = (gcol_c.astype(jnp.float32) == pr_s[...]).astype(jnp.bfloat16)
        valid = (j * BC + lax.broadcasted_iota(jnp.int32, (1, BC), 1)) < C
        for g in range(NG):
            wb = jnp.where(valid, w_refs[g][...], 0.0).astype(jnp.bfloat16)
            _gather_update(g, wb, oh)

        eye = (lax.broadcasted_iota(jnp.int32, (B, B), 0) ==
               lax.broadcasted_iota(jnp.int32, (B, B), 1)).astype(jnp.float32)
        wi = wij_s[:, :B]
        wj = wij_s[:, B:]
        mrow = jnp.sum(wi * wj, axis=0, keepdims=True)
        mcol = lax.dot_general(eye, mrow, (((1,), (1,)), ((), ())),
                               preferred_element_type=jnp.float32)
        m = m_s[...]
        t = t_s[...]
        zz = s_s[...] - jnp.exp(t - m) + jnp.exp(t - mcol - m)
        lossv = m + jnp.log(zz) - t + mcol
        o_ref[...] = jnp.sum(lossv, axis=0, keepdims=True) * (1.0 / B)


_fused = pl.pallas_call(
    _fused_body,
    grid=(2 * NB,),
    in_specs=(
        [pl.BlockSpec((B, 1), lambda i: (0, 0)),
         pl.BlockSpec((1, B), lambda i: (0, 0))] +
        [pl.BlockSpec((RG, BC), lambda i, g=g: (g, jnp.minimum(i, NB - 1)))
         for g in range(NG)] +
        [pl.BlockSpec((RG, BC), lambda i, g=g: (g, jnp.maximum(i - NB, 0)))
         for g in range(NG)]
    ),
    out_specs=pl.BlockSpec((1, 1), lambda i: (0, 0)),
    out_shape=jax.ShapeDtypeStruct((1, 1), jnp.float32),
    scratch_shapes=[
        pltpu.VMEM((B, 1), jnp.float32),
        pltpu.VMEM((B, 1), jnp.float32),
        pltpu.VMEM((B, 1), jnp.int32),
        pltpu.VMEM((B, 1), jnp.float32),
        pltpu.VMEM((1, 2 * B), jnp.float32),
        pltpu.VMEM((D, 2 * B), jnp.float32),
    ],
    compiler_params=pltpu.CompilerParams(
        dimension_semantics=("arbitrary",)),
)


@jax.jit
def kernel(y_pred, y_true, W):
    y_true = y_true.astype(jnp.int32)
    ytc = y_true.reshape(B, 1)
    ytr = y_true.astype(jnp.float32).reshape(1, B)
    loss = _fused(ytc, ytr,
                  *([y_pred] * NG), *([W] * NG))
    return loss.reshape(())
